# baseline (speedup 1.0000x reference)
import functools

import jax
import jax.numpy as jnp
from jax import lax
from jax.experimental import pallas as pl
from jax.experimental.pallas import tpu as pltpu
from jax.experimental.pallas import tpu_sc as plsc

BATCH = 16384
D = 64
L = 16


def _mf_call(src, dst, user_emb, user_bias_flat, item_emb, item_bias_flat, mean):
    info = plsc.get_sparse_core_info()
    nw = info.num_cores * info.num_subcores
    bw = BATCH // nw
    nblk = bw // L

    mesh = plsc.VectorSubcoreMesh(core_axis_name="c", subcore_axis_name="s")

    @functools.partial(
        pl.kernel,
        out_type=jax.ShapeDtypeStruct((BATCH,), jnp.float32),
        mesh=mesh,
        compiler_params=pltpu.CompilerParams(
            needs_layout_passes=False, use_tc_tiling_on_sc=False),
        scratch_types=[
            pltpu.VMEM((bw,), jnp.int32),
            pltpu.VMEM((bw,), jnp.int32),
            pltpu.VMEM((bw, D), jnp.float32),
            pltpu.VMEM((bw, D), jnp.float32),
            pltpu.VMEM((bw,), jnp.float32),
            pltpu.VMEM((bw,), jnp.float32),
            pltpu.VMEM((bw,), jnp.float32),
            pltpu.VMEM((L, L), jnp.float32),
            pltpu.VMEM((L,), jnp.float32),
            pltpu.SemaphoreType.DMA,
            pltpu.SemaphoreType.DMA,
            pltpu.SemaphoreType.DMA,
            pltpu.SemaphoreType.DMA,
        ],
    )
    def mf_kernel(src_hbm, dst_hbm, ue_hbm, ub_hbm, ie_hbm, ib_hbm, mean_hbm,
                  out_hbm, sidx, didx, urows, vrows, ub_v, ib_v, out_v, tscr,
                  mean_v, sem_u, sem_v, sem_ub, sem_ib):
        wid = lax.axis_index("s") * info.num_cores + lax.axis_index("c")
        base = wid * bw

        pltpu.sync_copy(src_hbm.at[pl.ds(base, bw)], sidx)
        pltpu.sync_copy(dst_hbm.at[pl.ds(base, bw)], didx)
        cu = pltpu.async_copy(ue_hbm.at[sidx], urows, sem_u)
        cv = pltpu.async_copy(ie_hbm.at[didx], vrows, sem_v)
        cub = pltpu.async_copy(ub_hbm.at[sidx], ub_v, sem_ub)
        cib = pltpu.async_copy(ib_hbm.at[didx], ib_v, sem_ib)
        pltpu.sync_copy(mean_hbm, mean_v)
        cu.wait()
        cv.wait()
        cub.wait()
        cib.wait()

        m = mean_v[...]
        lane = jnp.arange(L, dtype=jnp.int32)

        def blk_body(b, carry):
            rb = b * L
            for r in range(L):
                row = rb + r
                s = urows[row, pl.ds(0, L)] * vrows[row, pl.ds(0, L)]
                for c in range(1, D // L):
                    s = s + urows[row, pl.ds(c * L, L)] * vrows[row, pl.ds(c * L, L)]
                plsc.store_scatter(
                    tscr, [lane, jnp.full((L,), r, jnp.int32)], s)
            acc = tscr[0, :]
            for j in range(1, L):
                acc = acc + tscr[j, :]
            out_v[pl.ds(rb, L)] = (
                acc + ub_v[pl.ds(rb, L)] + ib_v[pl.ds(rb, L)] + m)
            return carry

        lax.fori_loop(0, nblk, blk_body, 0)
        pltpu.sync_copy(out_v, out_hbm.at[pl.ds(base, bw)])

    return mf_kernel(src, dst, user_emb, user_bias_flat, item_emb,
                     item_bias_flat, mean)


def kernel(src, dst, user_emb, user_bias, item_emb, item_bias, mean):
    return _mf_call(
        src.astype(jnp.int32),
        dst.astype(jnp.int32),
        user_emb,
        user_bias.reshape(-1),
        item_emb,
        item_bias.reshape(-1),
        jnp.broadcast_to(mean, (L,)),
    )


# speedup vs baseline: 1.3967x; 1.3967x over previous
import functools

import jax
import jax.numpy as jnp
from jax import lax
from jax.experimental import pallas as pl
from jax.experimental.pallas import tpu as pltpu
from jax.experimental.pallas import tpu_sc as plsc

BATCH = 16384
D = 64
L = 16
CH = 128
NBUF = 3


def _mf_call(src, dst, user_emb, user_bias_flat, item_emb, item_bias_flat, mean):
    info = plsc.get_sparse_core_info()
    nw = info.num_cores * info.num_subcores
    bw = BATCH // nw
    nch = bw // CH

    mesh = plsc.VectorSubcoreMesh(core_axis_name="c", subcore_axis_name="s")

    @functools.partial(
        pl.kernel,
        out_type=jax.ShapeDtypeStruct((BATCH,), jnp.float32),
        mesh=mesh,
        compiler_params=pltpu.CompilerParams(
            needs_layout_passes=False, use_tc_tiling_on_sc=True),
        scratch_types=[
            pltpu.VMEM((bw,), jnp.int32),
            pltpu.VMEM((bw,), jnp.int32),
            [pltpu.VMEM((CH, D), jnp.float32) for _ in range(NBUF)],
            [pltpu.VMEM((CH, D), jnp.float32) for _ in range(NBUF)],
            pltpu.VMEM((bw,), jnp.float32),
            pltpu.VMEM((bw,), jnp.float32),
            pltpu.VMEM((bw,), jnp.float32),
            pltpu.VMEM((L * L,), jnp.float32),
            pltpu.VMEM((L,), jnp.float32),
            [pltpu.SemaphoreType.DMA for _ in range(NBUF)],
            [pltpu.SemaphoreType.DMA for _ in range(NBUF)],
            pltpu.SemaphoreType.DMA,
            pltpu.SemaphoreType.DMA,
        ],
    )
    def mf_kernel(src_hbm, dst_hbm, ue_hbm, ub_hbm, ie_hbm, ib_hbm, mean_hbm,
                  out_hbm, sidx_v, didx_v, u_bufs, v_bufs,
                  ub_v, ib_v, out_v, tscr, mean_v, sems_u, sems_v, sem_ub,
                  sem_ib):
        wid = lax.axis_index("s") * info.num_cores + lax.axis_index("c")
        base = wid * bw

        pltpu.sync_copy(src_hbm.at[pl.ds(base, bw)], sidx_v)
        pltpu.sync_copy(dst_hbm.at[pl.ds(base, bw)], didx_v)
        cub = pltpu.async_copy(ub_hbm.at[sidx_v], ub_v, sem_ub)
        cib = pltpu.async_copy(ib_hbm.at[didx_v], ib_v, sem_ib)

        def issue(ch):
            sl = ch % NBUF

            def dma_body(g, carry):
                off = ch * CH + g * L
                sv = sidx_v[pl.ds(off, L)]
                dv = didx_v[pl.ds(off, L)]
                for j in range(L):
                    i = g * L + j
                    pltpu.async_copy(ue_hbm.at[sv[j]], u_bufs[sl].at[i],
                                     sems_u[sl])
                    pltpu.async_copy(ie_hbm.at[dv[j]], v_bufs[sl].at[i],
                                     sems_v[sl])
                return carry

            lax.fori_loop(0, CH // L, dma_body, 0)

        def drain(ch):
            sl = ch % NBUF

            def wait_body(i, carry):
                pltpu.make_async_copy(ue_hbm.at[0], u_bufs[sl].at[i],
                                      sems_u[sl]).wait()
                pltpu.make_async_copy(ie_hbm.at[0], v_bufs[sl].at[i],
                                      sems_v[sl]).wait()
                return carry

            lax.fori_loop(0, CH, wait_body, 0, unroll=4)

        lane16 = jnp.arange(L, dtype=jnp.int32) * L

        def compute(ch, m):
            sl = ch % NBUF

            def blk_body(b, carry):
                rb = b * L
                for r in range(L):
                    row = rb + r
                    s = (u_bufs[sl][row, pl.ds(0, L)]
                         * v_bufs[sl][row, pl.ds(0, L)])
                    for c in range(1, D // L):
                        s = s + (u_bufs[sl][row, pl.ds(c * L, L)]
                                 * v_bufs[sl][row, pl.ds(c * L, L)])
                    plsc.store_scatter(tscr, [lane16 + r], s)
                acc = tscr[pl.ds(0, L)]
                for j in range(1, L):
                    acc = acc + tscr[pl.ds(j * L, L)]
                ob = ch * CH + rb
                out_v[pl.ds(ob, L)] = (
                    acc + ub_v[pl.ds(ob, L)] + ib_v[pl.ds(ob, L)] + m)
                return carry

            lax.fori_loop(0, CH // L, blk_body, 0)

        issue(0)
        issue(1)
        pltpu.sync_copy(mean_hbm, mean_v)
        cub.wait()
        cib.wait()
        m = mean_v[...]
        for ch in range(nch):
            drain(ch)
            if ch + 2 < nch:
                issue(ch + 2)
            compute(ch, m)

        pltpu.sync_copy(out_v, out_hbm.at[pl.ds(base, bw)])

    return mf_kernel(src, dst, user_emb, user_bias_flat, item_emb,
                     item_bias_flat, mean)


def kernel(src, dst, user_emb, user_bias, item_emb, item_bias, mean):
    return _mf_call(
        src.astype(jnp.int32),
        dst.astype(jnp.int32),
        user_emb,
        user_bias.reshape(-1),
        item_emb,
        item_bias.reshape(-1),
        jnp.broadcast_to(mean, (L,)),
    )
